# Initial kernel scaffold; baseline (speedup 1.0000x reference)
import jax
import jax.numpy as jnp
from jax.experimental import pallas as pl


def kernel(gcn_vectors, token_types, obj_idx, sub_ptr, obj_ptr, W1, b1, W2, b2, abs_pos_emb, type_emb, self_idx_emb, sub_ptr_emb, obj_ptr_emb, rel_dist_emb):
    raise NotImplementedError("write your pallas kernel here")


# speedup vs baseline: 4.0356x; 4.0356x over previous
import functools

import jax
import jax.numpy as jnp
from jax import lax
from jax.experimental import pallas as pl

B, S = 1024, 77
GCN_DIM, MODEL_DIM = 512, 768
MAX_OBJS, MAX_SEQ_LEN = 100, 77
MAX_DIST = MAX_OBJS

N = B * S
BN = 512
G = N // BN

OFF_ABS = 0
OFF_SELF = OFF_ABS + MAX_SEQ_LEN
OFF_SUB = OFF_SELF + MAX_OBJS
OFF_OBJ = OFF_SUB + MAX_OBJS
OFF_REL = OFF_OBJ + MAX_OBJS
T_ROWS = OFF_REL + (2 * MAX_DIST + 1)
T_PAD = ((T_ROWS + 7) // 8) * 8


def _body(x_ref, idx_ref, w1_ref, b1_ref, w2_ref, b2_ref, tbl_ref, type_ref,
          xc_ref, xm_ref):
    x = x_ref[...]
    h = jnp.dot(x, w1_ref[...], preferred_element_type=jnp.float32) + b1_ref[...]
    h = 0.5 * h * (1.0 + lax.erf(h * 0.7071067811865476))
    xc = jnp.dot(h, w2_ref[...], preferred_element_type=jnp.float32) + b2_ref[...]
    xc_ref[...] = xc

    idx = idx_ref[0]
    pos = idx[:, 0:1]
    tok = idx[:, 1:2]
    oi = jnp.minimum(idx[:, 2:3], MAX_OBJS - 1)
    sp = jnp.minimum(idx[:, 3:4], MAX_OBJS - 1)
    op = jnp.minimum(idx[:, 4:5], MAX_OBJS - 1)
    ds = jnp.clip(pos - sp, -MAX_DIST, MAX_DIST) + MAX_DIST
    do = jnp.clip(pos - op, -MAX_DIST, MAX_DIST) + MAX_DIST
    t0 = tok == 0
    t1 = tok == 1

    c = lax.broadcasted_iota(jnp.int32, (BN, T_PAD), 1)
    u = (c == pos).astype(jnp.float32)
    u += ((c == oi + OFF_SELF) & t0).astype(jnp.float32)
    u += ((c == sp + OFF_SUB) & t1).astype(jnp.float32)
    u += ((c == op + OFF_OBJ) & t1).astype(jnp.float32)
    u += ((c == ds + OFF_REL) & t1).astype(jnp.float32)
    u += ((c == do + OFF_REL) & t1).astype(jnp.float32)
    emb = jnp.dot(u, tbl_ref[...], preferred_element_type=jnp.float32)

    tf = tok.astype(jnp.float32)
    typ = type_ref[0:1, :] + tf * (type_ref[1:2, :] - type_ref[0:1, :])
    xm_ref[...] = xc + emb + typ


@functools.partial(jax.jit, static_argnames=())
def kernel(gcn_vectors, token_types, obj_idx, sub_ptr, obj_ptr, W1, b1, W2, b2,
           abs_pos_emb, type_emb, self_idx_emb, sub_ptr_emb, obj_ptr_emb,
           rel_dist_emb):
    xg = gcn_vectors.reshape(N, GCN_DIM)
    pos = jnp.broadcast_to(
        jnp.minimum(jnp.arange(S, dtype=jnp.int32), MAX_SEQ_LEN - 1)[None, :],
        (B, S)).reshape(N)
    idx = jnp.stack(
        [pos,
         token_types.reshape(N).astype(jnp.int32),
         obj_idx.reshape(N).astype(jnp.int32),
         sub_ptr.reshape(N).astype(jnp.int32),
         obj_ptr.reshape(N).astype(jnp.int32)],
        axis=-1).reshape(G, BN, 5)
    tbl = jnp.concatenate(
        [abs_pos_emb, self_idx_emb, sub_ptr_emb, obj_ptr_emb, rel_dist_emb,
         jnp.zeros((T_PAD - T_ROWS, MODEL_DIM), jnp.float32)], axis=0)

    xc, xm = pl.pallas_call(
        _body,
        grid=(G,),
        in_specs=[
            pl.BlockSpec((BN, GCN_DIM), lambda i: (i, 0)),
            pl.BlockSpec((1, BN, 5), lambda i: (i, 0, 0)),
            pl.BlockSpec((GCN_DIM, MODEL_DIM), lambda i: (0, 0)),
            pl.BlockSpec((1, MODEL_DIM), lambda i: (0, 0)),
            pl.BlockSpec((MODEL_DIM, MODEL_DIM), lambda i: (0, 0)),
            pl.BlockSpec((1, MODEL_DIM), lambda i: (0, 0)),
            pl.BlockSpec((T_PAD, MODEL_DIM), lambda i: (0, 0)),
            pl.BlockSpec((2, MODEL_DIM), lambda i: (0, 0)),
        ],
        out_specs=[
            pl.BlockSpec((BN, MODEL_DIM), lambda i: (i, 0)),
            pl.BlockSpec((BN, MODEL_DIM), lambda i: (i, 0)),
        ],
        out_shape=[
            jax.ShapeDtypeStruct((N, MODEL_DIM), jnp.float32),
            jax.ShapeDtypeStruct((N, MODEL_DIM), jnp.float32),
        ],
    )(xg, idx, W1, b1.reshape(1, MODEL_DIM), W2, b2.reshape(1, MODEL_DIM),
      tbl, type_emb)
    return (xc.reshape(B, S, MODEL_DIM), xm.reshape(B, S, MODEL_DIM))


# speedup vs baseline: 4.1011x; 1.0163x over previous
import functools

import jax
import jax.numpy as jnp
from jax import lax
from jax.experimental import pallas as pl

B, S = 1024, 77
GCN_DIM, MODEL_DIM = 512, 768
MAX_OBJS, MAX_SEQ_LEN = 100, 77
MAX_DIST = MAX_OBJS

N = B * S
BN = 512
G = N // BN

OFF_ABS = 0
OFF_SELF = OFF_ABS + MAX_SEQ_LEN
OFF_SUB = OFF_SELF + MAX_OBJS
OFF_OBJ = OFF_SUB + MAX_OBJS
OFF_REL = OFF_OBJ + MAX_OBJS
T_ROWS = OFF_REL + (2 * MAX_DIST + 1)
T_PAD = ((T_ROWS + 7) // 8) * 8


def _body(x_ref, idx_ref, w1_ref, b1_ref, w2_ref, b2_ref, tbl_ref, type_ref,
          xc_ref, xm_ref):
    x = x_ref[...].astype(jnp.bfloat16)
    h = jnp.dot(x, w1_ref[...], preferred_element_type=jnp.float32) + b1_ref[...]
    h = 0.5 * h * (1.0 + lax.erf(h * 0.7071067811865476))
    xc = jnp.dot(h.astype(jnp.bfloat16), w2_ref[...],
                 preferred_element_type=jnp.float32) + b2_ref[...]
    xc_ref[...] = xc

    idx = idx_ref[0]
    pos = idx[:, 0:1]
    tok = idx[:, 1:2]
    oi = jnp.minimum(idx[:, 2:3], MAX_OBJS - 1)
    sp = jnp.minimum(idx[:, 3:4], MAX_OBJS - 1)
    op = jnp.minimum(idx[:, 4:5], MAX_OBJS - 1)
    ds = jnp.clip(pos - sp, -MAX_DIST, MAX_DIST) + MAX_DIST
    do = jnp.clip(pos - op, -MAX_DIST, MAX_DIST) + MAX_DIST
    t0 = tok == 0
    t1 = tok == 1

    c = lax.broadcasted_iota(jnp.int32, (BN, T_PAD), 1)
    u = (c == pos).astype(jnp.float32)
    u += ((c == oi + OFF_SELF) & t0).astype(jnp.float32)
    u += ((c == sp + OFF_SUB) & t1).astype(jnp.float32)
    u += ((c == op + OFF_OBJ) & t1).astype(jnp.float32)
    u += ((c == ds + OFF_REL) & t1).astype(jnp.float32)
    u += ((c == do + OFF_REL) & t1).astype(jnp.float32)
    emb = jnp.dot(u.astype(jnp.bfloat16), tbl_ref[...],
                  preferred_element_type=jnp.float32)

    tf = tok.astype(jnp.float32)
    typ = type_ref[0:1, :] + tf * (type_ref[1:2, :] - type_ref[0:1, :])
    xm_ref[...] = xc + emb + typ


@functools.partial(jax.jit, static_argnames=())
def kernel(gcn_vectors, token_types, obj_idx, sub_ptr, obj_ptr, W1, b1, W2, b2,
           abs_pos_emb, type_emb, self_idx_emb, sub_ptr_emb, obj_ptr_emb,
           rel_dist_emb):
    xg = gcn_vectors.reshape(N, GCN_DIM)
    pos = jnp.broadcast_to(
        jnp.minimum(jnp.arange(S, dtype=jnp.int32), MAX_SEQ_LEN - 1)[None, :],
        (B, S)).reshape(N)
    idx = jnp.stack(
        [pos,
         token_types.reshape(N).astype(jnp.int32),
         obj_idx.reshape(N).astype(jnp.int32),
         sub_ptr.reshape(N).astype(jnp.int32),
         obj_ptr.reshape(N).astype(jnp.int32)],
        axis=-1).reshape(G, BN, 5)
    tbl = jnp.concatenate(
        [abs_pos_emb, self_idx_emb, sub_ptr_emb, obj_ptr_emb, rel_dist_emb,
         jnp.zeros((T_PAD - T_ROWS, MODEL_DIM), jnp.float32)],
        axis=0).astype(jnp.bfloat16)

    xc, xm = pl.pallas_call(
        _body,
        grid=(G,),
        in_specs=[
            pl.BlockSpec((BN, GCN_DIM), lambda i: (i, 0)),
            pl.BlockSpec((1, BN, 5), lambda i: (i, 0, 0)),
            pl.BlockSpec((GCN_DIM, MODEL_DIM), lambda i: (0, 0)),
            pl.BlockSpec((1, MODEL_DIM), lambda i: (0, 0)),
            pl.BlockSpec((MODEL_DIM, MODEL_DIM), lambda i: (0, 0)),
            pl.BlockSpec((1, MODEL_DIM), lambda i: (0, 0)),
            pl.BlockSpec((T_PAD, MODEL_DIM), lambda i: (0, 0)),
            pl.BlockSpec((2, MODEL_DIM), lambda i: (0, 0)),
        ],
        out_specs=[
            pl.BlockSpec((BN, MODEL_DIM), lambda i: (i, 0)),
            pl.BlockSpec((BN, MODEL_DIM), lambda i: (i, 0)),
        ],
        out_shape=[
            jax.ShapeDtypeStruct((N, MODEL_DIM), jnp.float32),
            jax.ShapeDtypeStruct((N, MODEL_DIM), jnp.float32),
        ],
    )(xg, idx, W1.astype(jnp.bfloat16), b1.reshape(1, MODEL_DIM),
      W2.astype(jnp.bfloat16), b2.reshape(1, MODEL_DIM), tbl, type_emb)
    return (xc.reshape(B, S, MODEL_DIM), xm.reshape(B, S, MODEL_DIM))


# speedup vs baseline: 4.9997x; 1.2191x over previous
import functools

import jax
import jax.numpy as jnp
from jax import lax
from jax.experimental import pallas as pl

B, S = 1024, 77
GCN_DIM, MODEL_DIM = 512, 768
MAX_OBJS, MAX_SEQ_LEN = 100, 77
MAX_DIST = MAX_OBJS

BB = 8
G = B // BB

OFF_ABS = 0
OFF_SELF = OFF_ABS + MAX_SEQ_LEN
OFF_SUB = OFF_SELF + MAX_OBJS
OFF_OBJ = OFF_SUB + MAX_OBJS
OFF_REL = OFF_OBJ + MAX_OBJS
OFF_TYPE = OFF_REL + (2 * MAX_DIST + 1)
T_ROWS = OFF_TYPE + 2
T_PAD = ((T_ROWS + 7) // 8) * 8


def _body(x_ref, tt_ref, oi_ref, sp_ref, op_ref, w1_ref, b1_ref, w2_ref,
          b2_ref, tbl_ref, xc_ref, xm_ref):
    r_iota = lax.broadcasted_iota(jnp.int32, (T_PAD, S), 0)
    pos = lax.broadcasted_iota(jnp.int32, (1, S), 1)
    for b in range(BB):
        x = x_ref[b].astype(jnp.bfloat16)
        h = jnp.dot(x, w1_ref[...], preferred_element_type=jnp.float32)
        h = h + b1_ref[...]
        h = 0.5 * h * (1.0 + lax.erf(h * 0.7071067811865476))
        xc = jnp.dot(h.astype(jnp.bfloat16), w2_ref[...],
                     preferred_element_type=jnp.float32) + b2_ref[...]
        xc_ref[b] = xc

        tok = jnp.clip(tt_ref[b : b + 1, :], 0, 1)
        oi = jnp.minimum(oi_ref[b : b + 1, :], MAX_OBJS - 1)
        sp = jnp.minimum(sp_ref[b : b + 1, :], MAX_OBJS - 1)
        op = jnp.minimum(op_ref[b : b + 1, :], MAX_OBJS - 1)
        ds = jnp.clip(pos - sp, -MAX_DIST, MAX_DIST) + MAX_DIST
        do = jnp.clip(pos - op, -MAX_DIST, MAX_DIST) + MAX_DIST
        t0 = tok == 0
        t1 = tok == 1

        ut = (r_iota == pos).astype(jnp.float32)
        ut += (r_iota == tok + OFF_TYPE).astype(jnp.float32)
        ut += ((r_iota == oi + OFF_SELF) & t0).astype(jnp.float32)
        ut += ((r_iota == sp + OFF_SUB) & t1).astype(jnp.float32)
        ut += ((r_iota == op + OFF_OBJ) & t1).astype(jnp.float32)
        ut += ((r_iota == ds + OFF_REL) & t1).astype(jnp.float32)
        ut += ((r_iota == do + OFF_REL) & t1).astype(jnp.float32)
        emb = lax.dot_general(ut.astype(jnp.bfloat16), tbl_ref[...],
                              (((0,), (0,)), ((), ())),
                              preferred_element_type=jnp.float32)
        xm_ref[b] = xc + emb


@functools.partial(jax.jit, static_argnames=())
def kernel(gcn_vectors, token_types, obj_idx, sub_ptr, obj_ptr, W1, b1, W2, b2,
           abs_pos_emb, type_emb, self_idx_emb, sub_ptr_emb, obj_ptr_emb,
           rel_dist_emb):
    tbl = jnp.concatenate(
        [abs_pos_emb, self_idx_emb, sub_ptr_emb, obj_ptr_emb, rel_dist_emb,
         type_emb, jnp.zeros((T_PAD - T_ROWS, MODEL_DIM), jnp.float32)],
        axis=0).astype(jnp.bfloat16)

    xc, xm = pl.pallas_call(
        _body,
        grid=(G,),
        in_specs=[
            pl.BlockSpec((BB, S, GCN_DIM), lambda i: (i, 0, 0)),
            pl.BlockSpec((BB, S), lambda i: (i, 0)),
            pl.BlockSpec((BB, S), lambda i: (i, 0)),
            pl.BlockSpec((BB, S), lambda i: (i, 0)),
            pl.BlockSpec((BB, S), lambda i: (i, 0)),
            pl.BlockSpec((GCN_DIM, MODEL_DIM), lambda i: (0, 0)),
            pl.BlockSpec((1, MODEL_DIM), lambda i: (0, 0)),
            pl.BlockSpec((MODEL_DIM, MODEL_DIM), lambda i: (0, 0)),
            pl.BlockSpec((1, MODEL_DIM), lambda i: (0, 0)),
            pl.BlockSpec((T_PAD, MODEL_DIM), lambda i: (0, 0)),
        ],
        out_specs=[
            pl.BlockSpec((BB, S, MODEL_DIM), lambda i: (i, 0, 0)),
            pl.BlockSpec((BB, S, MODEL_DIM), lambda i: (i, 0, 0)),
        ],
        out_shape=[
            jax.ShapeDtypeStruct((B, S, MODEL_DIM), jnp.float32),
            jax.ShapeDtypeStruct((B, S, MODEL_DIM), jnp.float32),
        ],
    )(gcn_vectors, token_types.astype(jnp.int32), obj_idx.astype(jnp.int32),
      sub_ptr.astype(jnp.int32), obj_ptr.astype(jnp.int32),
      W1.astype(jnp.bfloat16), b1.reshape(1, MODEL_DIM),
      W2.astype(jnp.bfloat16), b2.reshape(1, MODEL_DIM), tbl)
    return (xc, xm)


# speedup vs baseline: 14.8563x; 2.9715x over previous
import functools

import jax
import jax.numpy as jnp
from jax import lax
from jax.experimental import pallas as pl

B, S = 1024, 77
GCN_DIM, MODEL_DIM = 512, 768
MAX_OBJS, MAX_SEQ_LEN = 100, 77
MAX_DIST = MAX_OBJS

OFF_SELF = 0
OFF_SUB = OFF_SELF + MAX_OBJS
OFF_OBJ = OFF_SUB + MAX_OBJS
OFF_REL = OFF_OBJ + MAX_OBJS
OFF_TYPE = OFF_REL + (2 * MAX_DIST + 1)
T_ROWS = OFF_TYPE + 2
T_PAD = ((T_ROWS + 7) // 8) * 8


def _body(x_ref, tt_ref, oi_ref, sp_ref, op_ref, w1_ref, b1_ref, w2_ref,
          b2_ref, tbl_ref, abs_ref, xc_ref, xm_ref):
    s = pl.program_id(0)
    x = x_ref[0].astype(jnp.bfloat16)
    h = jnp.dot(x, w1_ref[...], preferred_element_type=jnp.float32)
    h = h + b1_ref[...]
    h = 0.5 * h * (1.0 + lax.erf(h * 0.7071067811865476))
    xc = jnp.dot(h.astype(jnp.bfloat16), w2_ref[...],
                 preferred_element_type=jnp.float32) + b2_ref[...]
    xc_ref[0] = xc

    tok = jnp.clip(tt_ref[0], 0, 1)
    oi = jnp.minimum(oi_ref[0], MAX_OBJS - 1)
    sp = jnp.minimum(sp_ref[0], MAX_OBJS - 1)
    op = jnp.minimum(op_ref[0], MAX_OBJS - 1)
    ds = jnp.clip(s - sp, -MAX_DIST, MAX_DIST) + MAX_DIST
    do = jnp.clip(s - op, -MAX_DIST, MAX_DIST) + MAX_DIST
    t0 = tok == 0
    t1 = tok == 1

    r_iota = lax.broadcasted_iota(jnp.int32, (T_PAD, B), 0)
    ut = (r_iota == tok + OFF_TYPE).astype(jnp.float32)
    ut += ((r_iota == oi + OFF_SELF) & t0).astype(jnp.float32)
    ut += ((r_iota == sp + OFF_SUB) & t1).astype(jnp.float32)
    ut += ((r_iota == op + OFF_OBJ) & t1).astype(jnp.float32)
    ut += ((r_iota == ds + OFF_REL) & t1).astype(jnp.float32)
    ut += ((r_iota == do + OFF_REL) & t1).astype(jnp.float32)
    emb = lax.dot_general(ut.astype(jnp.bfloat16), tbl_ref[...],
                          (((0,), (0,)), ((), ())),
                          preferred_element_type=jnp.float32)
    xm_ref[0] = xc + emb + abs_ref[0]


@functools.partial(jax.jit, static_argnames=())
def kernel(gcn_vectors, token_types, obj_idx, sub_ptr, obj_ptr, W1, b1, W2, b2,
           abs_pos_emb, type_emb, self_idx_emb, sub_ptr_emb, obj_ptr_emb,
           rel_dist_emb):
    tbl = jnp.concatenate(
        [self_idx_emb, sub_ptr_emb, obj_ptr_emb, rel_dist_emb, type_emb,
         jnp.zeros((T_PAD - T_ROWS, MODEL_DIM), jnp.float32)],
        axis=0).astype(jnp.bfloat16)
    xg = jnp.transpose(gcn_vectors, (1, 0, 2))
    tt = jnp.transpose(token_types.astype(jnp.int32)).reshape(S, 1, B)
    oi = jnp.transpose(obj_idx.astype(jnp.int32)).reshape(S, 1, B)
    sp = jnp.transpose(sub_ptr.astype(jnp.int32)).reshape(S, 1, B)
    op = jnp.transpose(obj_ptr.astype(jnp.int32)).reshape(S, 1, B)

    xc, xm = pl.pallas_call(
        _body,
        grid=(S,),
        in_specs=[
            pl.BlockSpec((1, B, GCN_DIM), lambda i: (i, 0, 0)),
            pl.BlockSpec((1, 1, B), lambda i: (i, 0, 0)),
            pl.BlockSpec((1, 1, B), lambda i: (i, 0, 0)),
            pl.BlockSpec((1, 1, B), lambda i: (i, 0, 0)),
            pl.BlockSpec((1, 1, B), lambda i: (i, 0, 0)),
            pl.BlockSpec((GCN_DIM, MODEL_DIM), lambda i: (0, 0)),
            pl.BlockSpec((1, MODEL_DIM), lambda i: (0, 0)),
            pl.BlockSpec((MODEL_DIM, MODEL_DIM), lambda i: (0, 0)),
            pl.BlockSpec((1, MODEL_DIM), lambda i: (0, 0)),
            pl.BlockSpec((T_PAD, MODEL_DIM), lambda i: (0, 0)),
            pl.BlockSpec((1, 1, MODEL_DIM), lambda i: (i, 0, 0)),
        ],
        out_specs=[
            pl.BlockSpec((1, B, MODEL_DIM), lambda i: (i, 0, 0)),
            pl.BlockSpec((1, B, MODEL_DIM), lambda i: (i, 0, 0)),
        ],
        out_shape=[
            jax.ShapeDtypeStruct((S, B, MODEL_DIM), jnp.float32),
            jax.ShapeDtypeStruct((S, B, MODEL_DIM), jnp.float32),
        ],
    )(xg, tt, oi, sp, op,
      W1.astype(jnp.bfloat16), b1.reshape(1, MODEL_DIM),
      W2.astype(jnp.bfloat16), b2.reshape(1, MODEL_DIM), tbl,
      abs_pos_emb.reshape(S, 1, MODEL_DIM))
    return (jnp.transpose(xc, (1, 0, 2)), jnp.transpose(xm, (1, 0, 2)))


# speedup vs baseline: 17.7242x; 1.1930x over previous
import functools

import jax
import jax.numpy as jnp
from jax import lax
from jax.experimental import pallas as pl

B, S = 1024, 77
GCN_DIM, MODEL_DIM = 512, 768
MAX_OBJS, MAX_SEQ_LEN = 100, 77
MAX_DIST = MAX_OBJS

SEC_A, SEC_B, SEC_C, SEC_D = 208, 104, 208, 8
OFF_SUB_IN_A = 104
T_TOT = SEC_A + SEC_B + SEC_C + SEC_D


def _body(x_ref, tt_ref, oi_ref, sp_ref, op_ref, w1_ref, b1_ref, w2_ref,
          b2_ref, tbl_ref, abs_ref, xc_ref, xm_ref):
    s = pl.program_id(0)
    x = x_ref[0].astype(jnp.bfloat16)
    h = jnp.dot(x, w1_ref[...], preferred_element_type=jnp.float32)
    h = h + b1_ref[...]
    h = 0.5 * h * (1.0 + lax.erf(h * 0.7071067811865476))
    xc = jnp.dot(h.astype(jnp.bfloat16), w2_ref[...],
                 preferred_element_type=jnp.float32) + b2_ref[...]
    xc_ref[0] = xc

    tok = jnp.clip(tt_ref[0], 0, 1)
    oi = jnp.minimum(oi_ref[0], MAX_OBJS - 1)
    sp = jnp.minimum(sp_ref[0], MAX_OBJS - 1)
    op = jnp.minimum(op_ref[0], MAX_OBJS - 1)
    ds = jnp.clip(s - sp, -MAX_DIST, MAX_DIST) + MAX_DIST
    do = jnp.clip(s - op, -MAX_DIST, MAX_DIST) + MAX_DIST
    t0 = tok == 0
    t1 = tok == 1

    neg1 = jnp.full_like(tok, -1)
    idx_a = jnp.where(t0, oi, sp + OFF_SUB_IN_A)
    idx_b = jnp.where(t1, op, neg1)
    idx_c = jnp.where(t1, ds, neg1)
    idx_d = jnp.where(t1, do, neg1)
    ia = lax.broadcasted_iota(jnp.int32, (SEC_A, B), 0)
    ib = lax.broadcasted_iota(jnp.int32, (SEC_B, B), 0)
    ic = lax.broadcasted_iota(jnp.int32, (SEC_C, B), 0)
    it = lax.broadcasted_iota(jnp.int32, (SEC_D, B), 0)
    ua = (ia == idx_a).astype(jnp.bfloat16)
    ub = (ib == idx_b).astype(jnp.bfloat16)
    uc = ((ic == idx_c).astype(jnp.bfloat16)
          + (ic == idx_d).astype(jnp.bfloat16))
    ud = (it == tok).astype(jnp.bfloat16)
    ut = jnp.concatenate([ua, ub, uc, ud], axis=0)
    emb = lax.dot_general(ut, tbl_ref[...],
                          (((0,), (0,)), ((), ())),
                          preferred_element_type=jnp.float32)
    xm_ref[0] = xc + emb + abs_ref[0]


@functools.partial(jax.jit, static_argnames=())
def kernel(gcn_vectors, token_types, obj_idx, sub_ptr, obj_ptr, W1, b1, W2, b2,
           abs_pos_emb, type_emb, self_idx_emb, sub_ptr_emb, obj_ptr_emb,
           rel_dist_emb):
    z4 = jnp.zeros((4, MODEL_DIM), jnp.float32)
    tbl = jnp.concatenate(
        [self_idx_emb, z4, sub_ptr_emb, z4,
         obj_ptr_emb, z4,
         rel_dist_emb, jnp.zeros((7, MODEL_DIM), jnp.float32),
         type_emb, jnp.zeros((6, MODEL_DIM), jnp.float32)],
        axis=0).astype(jnp.bfloat16)
    xg = jnp.transpose(gcn_vectors, (1, 0, 2))
    tt = jnp.transpose(token_types.astype(jnp.int32)).reshape(S, 1, B)
    oi = jnp.transpose(obj_idx.astype(jnp.int32)).reshape(S, 1, B)
    sp = jnp.transpose(sub_ptr.astype(jnp.int32)).reshape(S, 1, B)
    op = jnp.transpose(obj_ptr.astype(jnp.int32)).reshape(S, 1, B)

    xc, xm = pl.pallas_call(
        _body,
        grid=(S,),
        in_specs=[
            pl.BlockSpec((1, B, GCN_DIM), lambda i: (i, 0, 0)),
            pl.BlockSpec((1, 1, B), lambda i: (i, 0, 0)),
            pl.BlockSpec((1, 1, B), lambda i: (i, 0, 0)),
            pl.BlockSpec((1, 1, B), lambda i: (i, 0, 0)),
            pl.BlockSpec((1, 1, B), lambda i: (i, 0, 0)),
            pl.BlockSpec((GCN_DIM, MODEL_DIM), lambda i: (0, 0)),
            pl.BlockSpec((1, MODEL_DIM), lambda i: (0, 0)),
            pl.BlockSpec((MODEL_DIM, MODEL_DIM), lambda i: (0, 0)),
            pl.BlockSpec((1, MODEL_DIM), lambda i: (0, 0)),
            pl.BlockSpec((T_TOT, MODEL_DIM), lambda i: (0, 0)),
            pl.BlockSpec((1, 1, MODEL_DIM), lambda i: (i, 0, 0)),
        ],
        out_specs=[
            pl.BlockSpec((1, B, MODEL_DIM), lambda i: (i, 0, 0)),
            pl.BlockSpec((1, B, MODEL_DIM), lambda i: (i, 0, 0)),
        ],
        out_shape=[
            jax.ShapeDtypeStruct((S, B, MODEL_DIM), jnp.float32),
            jax.ShapeDtypeStruct((S, B, MODEL_DIM), jnp.float32),
        ],
    )(xg, tt, oi, sp, op,
      W1.astype(jnp.bfloat16), b1.reshape(1, MODEL_DIM),
      W2.astype(jnp.bfloat16), b2.reshape(1, MODEL_DIM), tbl,
      abs_pos_emb.reshape(S, 1, MODEL_DIM))
    return (jnp.transpose(xc, (1, 0, 2)), jnp.transpose(xm, (1, 0, 2)))
